# baseline (speedup 1.0000x reference)
import jax
import jax.numpy as jnp
from jax import lax
from jax.experimental import pallas as pl
from jax.experimental.pallas import tpu as pltpu
from jax.experimental.pallas import tpu_sc as plsc

L_SEQ = 200
B_SEQ = 4096
D = 64
DP = 128
VOCAB = 1000000
G = D // 8
NBT = B_SEQ // 128
NC, NS = 2, 16
NW = NC * NS
LANES = 16
NBLK = 128 // LANES
NBUF = 2
N_PAIRS = L_SEQ // NBUF


def _emb_kernel(idx_hbm, table_hbm, out_hbm, idx_all, r0, r1, t0, t1,
                g0, g1, o0, o1):
    rows = [r0, r1]
    tiles = [t0, t1]
    gsem = [g0, g1]
    osem = [o0, o1]
    wid = lax.axis_index("s") * NC + lax.axis_index("c")
    col0 = wid * 128

    pltpu.sync_copy(idx_hbm.at[pl.ds(0, L_SEQ), pl.ds(col0, 128)], idx_all)

    iotas = [lax.iota(jnp.int32, LANES) + (blk * LANES) for blk in range(NBLK)]

    def fire(l, b):
        pltpu.async_copy(table_hbm.at[idx_all.at[l]], rows[b], gsem[b])

    def wait_gather(b):
        pltpu.make_async_copy(
            table_hbm.at[pl.ds(0, 128), :], rows[b], gsem[b]
        ).wait()

    def start_out(l, b):
        pltpu.async_copy(tiles[b], out_hbm.at[l, :, wid, :, :], osem[b])

    def wait_out(b):
        pltpu.make_async_copy(tiles[b], out_hbm.at[0, :, 0, :, :], osem[b]).wait()

    def transpose(b):
        rq = rows[b]
        tl = tiles[b]

        @plsc.parallel_loop(0, D, unroll=8)
        def _d(d):
            cols = jnp.full((LANES,), d, jnp.int32)
            gq = d // 8
            dd = d - gq * 8
            for blk in range(NBLK):
                v = plsc.load_gather(rq, [iotas[blk], cols])
                tl[gq, dd, pl.ds(blk * LANES, LANES)] = v

    for b in range(NBUF):
        fire(b, b)

    @pl.loop(0, N_PAIRS)
    def _pair(p):
        for b in range(NBUF):
            l = p * NBUF + b
            wait_gather(b)

            @pl.when(l >= NBUF)
            def _():
                wait_out(b)

            transpose(b)
            start_out(l, b)
            ln = l + NBUF

            @pl.when(ln < L_SEQ)
            def _():
                fire(ln, b)

    for b in range(NBUF):
        wait_out(b)


def kernel(sequence, W):
    W_pad = jnp.pad(W, ((0, 0), (0, DP - D)))
    W2 = W_pad.reshape(2 * VOCAB, D)
    seq2 = sequence * 2
    mesh = plsc.VectorSubcoreMesh(core_axis_name="c", subcore_axis_name="s")
    out5 = pl.kernel(
        _emb_kernel,
        out_type=jax.ShapeDtypeStruct((L_SEQ, G, NBT, 8, 128), jnp.float32),
        mesh=mesh,
        scratch_types=[
            pltpu.VMEM((L_SEQ, 128), jnp.int32),
            pltpu.VMEM((128, D), jnp.float32),
            pltpu.VMEM((128, D), jnp.float32),
            pltpu.VMEM((G, 8, 128), jnp.float32),
            pltpu.VMEM((G, 8, 128), jnp.float32),
            pltpu.SemaphoreType.DMA,
            pltpu.SemaphoreType.DMA,
            pltpu.SemaphoreType.DMA,
            pltpu.SemaphoreType.DMA,
        ],
        compiler_params=pltpu.CompilerParams(
            use_tc_tiling_on_sc=False, needs_layout_passes=False
        ),
    )(seq2, W2)
    return out5.transpose(0, 2, 4, 1, 3).reshape(L_SEQ, B_SEQ, D)


# speedup vs baseline: 1.4698x; 1.4698x over previous
import jax
import jax.numpy as jnp
from jax import lax
from jax.experimental import pallas as pl
from jax.experimental.pallas import tpu as pltpu
from jax.experimental.pallas import tpu_sc as plsc

L_SEQ = 200
B_SEQ = 4096
D = 64
DP = 128
VOCAB = 1000000
N_TOTAL = L_SEQ * B_SEQ
NC, NS = 2, 16
NW = NC * NS
IDX_MINOR = 128
K = 4
N_CHUNKS = L_SEQ // K
NBUF = 2
N_GROUPS = N_CHUNKS // NBUF


def _emb_kernel(idx_hbm, table_hbm, out_hbm, idx_all, r0, r1, g0, g1, o0, o1):
    rows = [r0, r1]
    gsem = [g0, g1]
    osem = [o0, o1]
    wid = lax.axis_index("s") * NC + lax.axis_index("c")
    col0 = wid * IDX_MINOR

    pltpu.sync_copy(idx_hbm.at[pl.ds(0, L_SEQ), pl.ds(col0, IDX_MINOR)], idx_all)

    def fire(c, b):
        for j in range(K):
            pltpu.async_copy(
                table_hbm.at[idx_all.at[c * K + j]],
                rows[b].at[j],
                gsem[b],
            )

    def wait_gathers(b):
        pltpu.make_async_copy(
            out_hbm.at[pl.ds(0, K * IDX_MINOR), pl.ds(0, D)], rows[b], gsem[b]
        ).wait()

    def start_out(c, b):
        for j in range(K):
            pltpu.async_copy(
                rows[b].at[j],
                out_hbm.at[pl.ds((c * K + j) * B_SEQ + col0, IDX_MINOR), pl.ds(0, D)],
                osem[b],
            )

    def wait_out(b):
        pltpu.make_async_copy(
            rows[b], out_hbm.at[pl.ds(0, K * IDX_MINOR), pl.ds(0, D)], osem[b]
        ).wait()

    for b in range(NBUF):
        fire(b, b)

    @pl.loop(0, N_GROUPS)
    def _group(g):
        for b in range(NBUF):
            c = g * NBUF + b
            wait_gathers(b)
            start_out(c, b)
        for b in range(NBUF):
            cn = g * NBUF + b + NBUF

            @pl.when(cn < N_CHUNKS)
            def _():
                wait_out(b)
                fire(cn, b)

    for b in range(NBUF):
        wait_out(b)


def kernel(sequence, W):
    W_pad = jnp.pad(jax.lax.optimization_barrier(W), ((0, 0), (0, DP - D)))
    W2 = W_pad.reshape(2 * VOCAB, D)
    seq2 = sequence * 2
    mesh = plsc.VectorSubcoreMesh(core_axis_name="c", subcore_axis_name="s")
    out_pad = pl.kernel(
        _emb_kernel,
        out_type=jax.ShapeDtypeStruct((N_TOTAL, DP), jnp.float32),
        mesh=mesh,
        scratch_types=[
            pltpu.VMEM((L_SEQ, IDX_MINOR), jnp.int32),
            pltpu.VMEM((K, IDX_MINOR, D), jnp.float32),
            pltpu.VMEM((K, IDX_MINOR, D), jnp.float32),
            pltpu.SemaphoreType.DMA,
            pltpu.SemaphoreType.DMA,
            pltpu.SemaphoreType.DMA,
            pltpu.SemaphoreType.DMA,
        ],
        compiler_params=pltpu.CompilerParams(use_tc_tiling_on_sc=False),
    )(seq2, W2)
    return out_pad[:, :D].reshape(L_SEQ, B_SEQ, D)
